# baseline (speedup 1.0000x reference)
import functools

import jax
import jax.numpy as jnp
from jax import lax
from jax.experimental import pallas as pl
from jax.experimental.pallas import tpu as pltpu
from jax.experimental.pallas import tpu_sc as plsc

N = 10000
E = 160000
D = 256
L = 3

NW = 32
RPT = 320
NPAD = NW * RPT
DUMMY = 313
CHUNK = E // NW
K = 64
CAP = 5120

_mesh = plsc.VectorSubcoreMesh(core_axis_name="c", subcore_axis_name="s")

_i32 = jnp.int32
_f32 = jnp.float32


@functools.partial(
    pl.kernel,
    out_type=(
        jax.ShapeDtypeStruct((NW * NW * 2 * CAP,), _i32),
        jax.ShapeDtypeStruct((NW * NW * CAP,), _i32),
        jax.ShapeDtypeStruct((NW * NW,), _i32),
    ),
    mesh=_mesh,
    compiler_params=pltpu.CompilerParams(needs_layout_passes=False),
    scratch_types=[
        pltpu.VMEM((CHUNK + 16,), _i32),
        pltpu.VMEM((CHUNK + 16,), _i32),
        pltpu.VMEM((2 * CAP,), _i32),
        pltpu.VMEM((CAP,), _i32),
        pltpu.VMEM((NW,), _i32),
    ],
)
def _route(dst_hbm, srci_hbm, cidx_out, dls_out, cnt_out,
           dstc, srcc, cst, dst_st, cvm):
    c = lax.axis_index("c")
    s = lax.axis_index("s")
    p = c * 16 + s
    ebase = p * CHUNK
    pltpu.sync_copy(dst_hbm.at[pl.ds(ebase, CHUNK)], dstc.at[pl.ds(0, CHUNK)])
    pltpu.sync_copy(srci_hbm.at[pl.ds(ebase, CHUNK)], srcc.at[pl.ds(0, CHUNK)])

    iota = lax.broadcasted_iota(_i32, (16,), 0)
    nv = (CHUNK + 15) // 16

    def bucket(b, bc):
        def zfill(i, cc):
            cst[pl.ds(i * 16, 16)] = jnp.zeros((16,), _i32)
            return cc

        lax.fori_loop(0, 2 * CAP // 16, zfill, 0)

        def dfill(i, cc):
            dst_st[pl.ds(i * 16, 16)] = jnp.full((16,), DUMMY, _i32)
            return cc

        lax.fori_loop(0, CAP // 16, dfill, 0)

        def scan(i, cntv):
            d = dstc[pl.ds(i * 16, 16)]
            m = ((d & 31) == b) & (i * 16 + iota < CHUNK)
            mi = m.astype(_i32)
            pos = cntv + plsc.cumsum(mi) - 1
            pos2 = pos + pos
            plsc.store_scatter(cst, [pos2], srcc[pl.ds(i * 16, 16)], mask=m)
            plsc.store_scatter(cst, [pos2 + 1],
                               (NPAD + ebase) + i * 16 + iota, mask=m)
            plsc.store_scatter(dst_st, [pos], d >> 5, mask=m)
            return cntv + plsc.all_reduce_population_count(m)

        cntv = lax.fori_loop(0, nv, scan, jnp.zeros((16,), _i32))

        roff = (b * NW + p) * CAP
        pltpu.sync_copy(cst, cidx_out.at[pl.ds(roff * 2, 2 * CAP)])
        pltpu.sync_copy(dst_st, dls_out.at[pl.ds(roff, CAP)])
        plsc.store_scatter(cvm, [iota * 0 + b], cntv, mask=iota == 0)
        return bc

    lax.fori_loop(0, NW, bucket, 0)
    pltpu.sync_copy(cvm, cnt_out.at[pl.ds(p * NW, NW)])


@functools.partial(
    pl.kernel,
    out_type=jax.ShapeDtypeStruct((NPAD, D), _f32),
    mesh=_mesh,
    compiler_params=pltpu.CompilerParams(needs_layout_passes=False),
    scratch_types=[
        pltpu.VMEM((RPT, D), _f32),
        pltpu.VMEM((2 * K, D), _f32),
        pltpu.VMEM((2 * CAP,), _i32),
        pltpu.VMEM((CAP,), _i32),
        pltpu.VMEM((16,), _i32),
    ],
)
def _segment(ct_hbm, cidx_hbm, dls_hbm, cnt_hbm, out_hbm,
             acc, buf, cib, dlb, cbuf):
    c = lax.axis_index("c")
    s = lax.axis_index("s")
    w = c * 16 + s
    iota = lax.broadcasted_iota(_i32, (16,), 0)
    zero16 = jnp.zeros((16,), _f32)

    def zrow(r, cc):
        for g in range(D // 16):
            acc[r, pl.ds(g * 16, 16)] = zero16
        return cc

    lax.fori_loop(0, RPT, zrow, 0)

    lane = w & 15
    half = w & 16

    def region(p, pc):
        pltpu.sync_copy(cnt_hbm.at[pl.ds(pl.multiple_of(p * NW + half, 16), 16)],
                        cbuf)
        cvec = cbuf[pl.ds(0, 16)]
        cnt = jnp.sum(jnp.where(iota == lane, cvec, 0))
        nch = (cnt + (K - 1)) >> 6
        roff = pl.multiple_of((w * NW + p) * CAP, 8)
        pltpu.sync_copy(cidx_hbm.at[pl.ds(roff * 2, 2 * CAP)], cib)
        pltpu.sync_copy(dls_hbm.at[pl.ds(roff, CAP)], dlb)

        def chunk(i, cc):
            off2 = pl.multiple_of(i * (2 * K), 2 * K)
            pltpu.sync_copy(ct_hbm.at[cib.at[pl.ds(off2, 2 * K)]], buf)

            def edge16(j, ec):
                dlv = dlb[pl.ds(pl.multiple_of(i * K + j * 16, 16), 16)]
                for e0 in range(0, 16, 4):
                    rows = [dlv[e0 + t] for t in range(4)]
                    for g0 in range(0, D // 16, 4):
                        va = [buf[j * 32 + 2 * (e0 + t), pl.ds((g0 + q) * 16, 16)]
                              for t in range(4) for q in range(4)]
                        vb = [buf[j * 32 + 2 * (e0 + t) + 1,
                                  pl.ds((g0 + q) * 16, 16)]
                              for t in range(4) for q in range(4)]
                        vs = [jnp.maximum(a + b, 0.0) for a, b in zip(va, vb)]
                        for t in range(4):
                            for q in range(4):
                                plsc.addupdate(
                                    acc.at[rows[t], pl.ds((g0 + q) * 16, 16)],
                                    vs[t * 4 + q])
                return ec

            lax.fori_loop(0, K // 16, edge16, 0)
            return cc

        lax.fori_loop(0, nch, chunk, 0)
        return pc

    lax.fori_loop(0, NW, region, 0)
    pltpu.sync_copy(acc, out_hbm.at[pl.ds(w * RPT, RPT)])


def _mlp_body(agg_ref, x_ref, w1_ref, b1_ref, w2_ref, b2_ref, o_ref, *, final):
    h0 = agg_ref[...] + x_ref[...]
    h = jnp.dot(h0, w1_ref[...], preferred_element_type=_f32) + b1_ref[...]
    h = jnp.maximum(h, 0.0)
    h2 = jnp.dot(h, w2_ref[...], preferred_element_type=_f32) + b2_ref[...]
    if not final:
        h2 = jnp.maximum(h2, 0.0)
    o_ref[...] = h2 + x_ref[...]


def _mlp_call(agg, x, w1, b1, w2, b2, final):
    B = 1024
    return pl.pallas_call(
        functools.partial(_mlp_body, final=final),
        grid=(NPAD // B,),
        in_specs=[
            pl.BlockSpec((B, D), lambda i: (i, 0)),
            pl.BlockSpec((B, D), lambda i: (i, 0)),
            pl.BlockSpec((D, D), lambda i: (0, 0)),
            pl.BlockSpec((1, D), lambda i: (0, 0)),
            pl.BlockSpec((D, D), lambda i: (0, 0)),
            pl.BlockSpec((1, D), lambda i: (0, 0)),
        ],
        out_specs=pl.BlockSpec((B, D), lambda i: (i, 0)),
        out_shape=jax.ShapeDtypeStruct((NPAD, D), _f32),
    )(agg, x, w1, b1, w2, b2)


def kernel(node_attr, edge_index, edge_attr, W1, b1, W2, b2):
    src = edge_index[0]
    dst = edge_index[1]
    src_i = (src & 31) * RPT + (src >> 5)
    pos = jnp.arange(NPAD, dtype=_i32)
    tile, r = pos // RPT, pos % RPT
    node = r * 32 + tile
    valid = (r < 313) & (node < N)
    xi = jnp.take(node_attr, jnp.where(valid, node, 0), axis=0)

    cidx, dls, counts = _route(dst, src_i)

    for l in range(L):
        ct = jnp.concatenate([xi, edge_attr], axis=0)
        agg = _segment(ct, cidx, dls, counts)
        xi = _mlp_call(agg, xi, W1[l], b1[l].reshape(1, D), W2[l],
                       b2[l].reshape(1, D), final=(l == L - 1))

    n = jnp.arange(N, dtype=_i32)
    return jnp.take(xi, (n & 31) * RPT + (n >> 5), axis=0)
